# baseline (speedup 1.0000x reference)
import functools

import jax
import jax.numpy as jnp
from jax import lax
from jax.experimental import pallas as pl
from jax.experimental.pallas import tpu as pltpu
from jax.experimental.pallas import tpu_sc as plsc

_NC = 2
_NS = 16
_NW = _NC * _NS
_CHUNK = 128


def _sc_gather(table, idx):
    B = idx.shape[0]
    D = table.shape[1]
    b_per_w = B // _NW
    n_chunks = b_per_w // _CHUNK

    mesh = plsc.VectorSubcoreMesh(core_axis_name="c", subcore_axis_name="s")

    @functools.partial(
        pl.kernel,
        mesh=mesh,
        out_type=jax.ShapeDtypeStruct((B, D), jnp.float32),
        scratch_types=[
            pltpu.VMEM((b_per_w,), jnp.int32),
            pltpu.VMEM((b_per_w, D), jnp.float32),
        ]
        + [pltpu.SemaphoreType.DMA] * (b_per_w // _CHUNK)
        + [pltpu.SemaphoreType.DMA],
    )
    def k(table_hbm, idx_hbm, out_hbm, idx_v, rows_v, *sems):
        gsems, ssem = sems[:-1], sems[-1]
        wid = lax.axis_index("s") * _NC + lax.axis_index("c")
        base = wid * b_per_w

        def gather(j):
            return pltpu.async_copy(
                table_hbm.at[idx_v.at[pl.ds(j * _CHUNK, _CHUNK)]],
                rows_v.at[pl.ds(j * _CHUNK, _CHUNK)],
                gsems[j],
            )

        pltpu.sync_copy(idx_hbm.at[pl.ds(base, b_per_w)], idx_v)
        gathers = [gather(0), gather(1)]
        stores = []
        for j in range(n_chunks):
            gathers[j].wait()
            stores.append(
                pltpu.async_copy(
                    rows_v.at[pl.ds(j * _CHUNK, _CHUNK)],
                    out_hbm.at[pl.ds(base + j * _CHUNK, _CHUNK)],
                    ssem,
                )
            )
            if j + 2 < n_chunks:
                gathers.append(gather(j + 2))
        for s in stores:
            s.wait()

    return k(table, idx)


def _proj_body(lang_ref, g_ref, lt_ref, wv_ref, wl_ref, b_ref, out_ref):
    g = g_ref[...]
    lang_proj = (
        jnp.dot(lt_ref[...], wl_ref[...], preferred_element_type=jnp.float32)
        + b_ref[...]
    )
    ids = lang_ref[0, 0]
    onehot = (
        ids.reshape(-1, 1) == lax.broadcasted_iota(jnp.int32, (1, lt_ref.shape[0]), 1)
    ).astype(jnp.float32)
    out_ref[...] = (
        jnp.dot(g, wv_ref[...], preferred_element_type=jnp.float32)
        + jnp.dot(onehot, lang_proj, preferred_element_type=jnp.float32)
    )


def kernel(voice_id, language_id, voice_table, lang_table, W, b):
    B = voice_id.shape[0]
    D = voice_table.shape[1]
    NL, LD = lang_table.shape

    gathered = _sc_gather(voice_table, voice_id.astype(jnp.int32))

    BLK = 8192
    grid = B // BLK
    lang3 = language_id.astype(jnp.int32).reshape(grid, 1, BLK)
    b2 = b.reshape(1, D)

    out = pl.pallas_call(
        _proj_body,
        grid=(grid,),
        in_specs=[
            pl.BlockSpec((1, 1, BLK), lambda i: (i, 0, 0)),
            pl.BlockSpec((BLK, D), lambda i: (i, 0)),
            pl.BlockSpec((NL, LD), lambda i: (0, 0)),
            pl.BlockSpec((D, D), lambda i: (0, 0)),
            pl.BlockSpec((LD, D), lambda i: (D // LD, 0)),
            pl.BlockSpec((1, D), lambda i: (0, 0)),
        ],
        out_specs=pl.BlockSpec((BLK, D), lambda i: (i, 0)),
        out_shape=jax.ShapeDtypeStruct((B, D), jnp.float32),
        compiler_params=pltpu.CompilerParams(
            dimension_semantics=("parallel",),
        ),
    )(lang3, gathered, lang_table, W, W, b2)
    return out


# speedup vs baseline: 1.0293x; 1.0293x over previous
import functools

import jax
import jax.numpy as jnp
from jax import lax
from jax.experimental import pallas as pl
from jax.experimental.pallas import tpu as pltpu
from jax.experimental.pallas import tpu_sc as plsc

_NC = 2
_NS = 16
_NW = _NC * _NS
_CHUNK = 128


def _sc_gather(table, idx):
    B = idx.shape[0]
    D = table.shape[1]
    b_per_w = B // _NW
    n_chunks = b_per_w // _CHUNK

    mesh = plsc.VectorSubcoreMesh(core_axis_name="c", subcore_axis_name="s")

    @functools.partial(
        pl.kernel,
        mesh=mesh,
        out_type=jax.ShapeDtypeStruct((B, D), jnp.float32),
        scratch_types=[
            pltpu.VMEM((b_per_w,), jnp.int32),
            pltpu.VMEM((b_per_w, D), jnp.float32),
        ]
        + [pltpu.SemaphoreType.DMA] * (b_per_w // _CHUNK)
        + [pltpu.SemaphoreType.DMA],
    )
    def k(table_hbm, idx_hbm, out_hbm, idx_v, rows_v, *sems):
        gsems, ssem = sems[:-1], sems[-1]
        wid = lax.axis_index("s") * _NC + lax.axis_index("c")
        base = wid * b_per_w

        def gather(j):
            return pltpu.async_copy(
                table_hbm.at[idx_v.at[pl.ds(j * _CHUNK, _CHUNK)]],
                rows_v.at[pl.ds(j * _CHUNK, _CHUNK)],
                gsems[j],
            )

        pltpu.sync_copy(idx_hbm.at[pl.ds(base, b_per_w)], idx_v)
        gathers = [gather(j) for j in range(n_chunks)]
        for g in gathers:
            g.wait()
        pltpu.sync_copy(rows_v, out_hbm.at[pl.ds(base, b_per_w)])
        del ssem

    return k(table, idx)


def _proj_body(lang_ref, g_ref, lt_ref, wv_ref, wl_ref, b_ref, out_ref):
    g = g_ref[...]
    lang_proj = (
        jnp.dot(lt_ref[...], wl_ref[...], preferred_element_type=jnp.float32)
        + b_ref[...]
    )
    ids = lang_ref[0, 0]
    onehot = (
        ids.reshape(-1, 1) == lax.broadcasted_iota(jnp.int32, (1, lt_ref.shape[0]), 1)
    ).astype(jnp.float32)
    out_ref[...] = (
        jnp.dot(
            g.astype(jnp.bfloat16),
            wv_ref[...].astype(jnp.bfloat16),
            preferred_element_type=jnp.float32,
        )
        + jnp.dot(onehot, lang_proj, preferred_element_type=jnp.float32)
    )


def kernel(voice_id, language_id, voice_table, lang_table, W, b):
    B = voice_id.shape[0]
    D = voice_table.shape[1]
    NL, LD = lang_table.shape

    gathered = _sc_gather(voice_table, voice_id.astype(jnp.int32))

    BLK = 8192
    grid = B // BLK
    lang3 = language_id.astype(jnp.int32).reshape(grid, 1, BLK)
    b2 = b.reshape(1, D)

    out = pl.pallas_call(
        _proj_body,
        grid=(grid,),
        in_specs=[
            pl.BlockSpec((1, 1, BLK), lambda i: (i, 0, 0)),
            pl.BlockSpec((BLK, D), lambda i: (i, 0)),
            pl.BlockSpec((NL, LD), lambda i: (0, 0)),
            pl.BlockSpec((D, D), lambda i: (0, 0)),
            pl.BlockSpec((LD, D), lambda i: (D // LD, 0)),
            pl.BlockSpec((1, D), lambda i: (0, 0)),
        ],
        out_specs=pl.BlockSpec((BLK, D), lambda i: (i, 0)),
        out_shape=jax.ShapeDtypeStruct((B, D), jnp.float32),
        compiler_params=pltpu.CompilerParams(
            dimension_semantics=("parallel",),
        ),
    )(lang3, gathered, lang_table, W, W, b2)
    return out
